# baseline (speedup 1.0000x reference)
import functools

import jax
import jax.numpy as jnp
from jax import lax
from jax.experimental import pallas as pl
from jax.experimental.pallas import tpu as pltpu
from jax.experimental.pallas import tpu_sc as plsc

DIM = 768
HID = 4 * DIM
EXPERTS = 8
T = 2048
BLK = 256
G = 16
SLOTS = G * BLK


def _router_body(x_ref, wr_ref, spos_ref, be_ref, bc_ref, nt_ref):
    x = x_ref[...]
    wr = wr_ref[...]
    scores = lax.dot_general(x, wr, (((1,), (1,)), ((), ())),
                             preferred_element_type=jnp.float32)
    best = scores[:, 0]
    eid = jnp.zeros((T,), jnp.int32)
    for e in range(1, EXPERTS):
        s = scores[:, e]
        upd = s > best
        eid = jnp.where(upd, e, eid)
        best = jnp.where(upd, s, best)

    onehot = (eid[:, None] == lax.broadcasted_iota(jnp.int32, (T, EXPERTS), 1))
    onehot = onehot.astype(jnp.float32)
    row = lax.broadcasted_iota(jnp.int32, (T, T), 0)
    col = lax.broadcasted_iota(jnp.int32, (T, T), 1)
    ltri = (row >= col).astype(jnp.float32)
    csum = lax.dot_general(ltri, onehot, (((1,), (0,)), ((), ())),
                           preferred_element_type=jnp.float32)
    rank = jnp.sum(csum * onehot, axis=1).astype(jnp.int32) - 1
    counts = jnp.sum(onehot, axis=0).astype(jnp.int32)

    nblocks = (counts + (BLK - 1)) // BLK
    e_row = lax.broadcasted_iota(jnp.int32, (EXPERTS, EXPERTS), 0)
    e_col = lax.broadcasted_iota(jnp.int32, (EXPERTS, EXPERTS), 1)
    gstart = jnp.sum(jnp.where(e_col < e_row, nblocks[None, :], 0), axis=1)
    nt = jnp.sum(nblocks)

    gstart_tok = jnp.sum(onehot * gstart[None, :].astype(jnp.float32), axis=1)
    spos_ref[...] = gstart_tok.astype(jnp.int32) * BLK + rank

    gi = lax.broadcasted_iota(jnp.int32, (G, EXPERTS), 0)
    ei = lax.broadcasted_iota(jnp.int32, (G, EXPERTS), 1)
    ind = (gi >= gstart[None, :]) & (gi < (gstart + nblocks)[None, :])
    be = jnp.sum(jnp.where(ind, ei, 0), axis=1)
    rows_left = counts[None, :] - (gi - gstart[None, :]) * BLK
    bc = jnp.sum(jnp.where(ind, jnp.clip(rows_left, 0, BLK), 0), axis=1)
    gvec = lax.iota(jnp.int32, G)
    lastmask = gvec == (nt - 1)
    be_last = jnp.sum(jnp.where(lastmask, be, 0))
    valid = gvec < nt
    be_ref[...] = jnp.where(valid, be, be_last)
    bc_ref[...] = jnp.where(valid, bc, 0)
    nt_ref[...] = jnp.full((1,), nt, jnp.int32)


def _ffn_body(be_s, bc_s, nt_s, inv_s,
              xs_ref, w1_ref, b1_ref, w2_ref, b2_ref, out_ref, y_scr):
    g = pl.program_id(0)

    @pl.when(g < nt_s[0])
    def _():
        xb = xs_ref[...]
        h = jnp.dot(xb, w1_ref[0], preferred_element_type=jnp.float32)
        h = h + b1_ref[0]
        h = 0.5 * h * (1.0 + lax.erf(h * 0.7071067811865476))
        y = jnp.dot(h, w2_ref[0], preferred_element_type=jnp.float32)
        y_scr[...] = y + b2_ref[0]

        def scatter_row(i, carry):
            t = inv_s[g * BLK + i]
            out_ref[pl.ds(t, 1), :] = y_scr[pl.ds(i, 1), :]
            return carry

        lax.fori_loop(0, bc_s[g], scatter_row, 0)


def _router(x, Wr):
    return pl.pallas_call(
        _router_body,
        out_shape=(
            jax.ShapeDtypeStruct((T,), jnp.int32),
            jax.ShapeDtypeStruct((G,), jnp.int32),
            jax.ShapeDtypeStruct((G,), jnp.int32),
            jax.ShapeDtypeStruct((1,), jnp.int32),
        ),
    )(x, Wr)


def _ffn(be, bc, nt, inv, xs, W1, b1, W2, b2):
    grid_spec = pltpu.PrefetchScalarGridSpec(
        num_scalar_prefetch=4,
        grid=(G,),
        in_specs=[
            pl.BlockSpec((BLK, DIM),
                         lambda g, be, bc, nt, inv: (jnp.minimum(g, nt[0] - 1), 0)),
            pl.BlockSpec((1, DIM, HID), lambda g, be, bc, nt, inv: (be[g], 0, 0)),
            pl.BlockSpec((1, 1, HID), lambda g, be, bc, nt, inv: (be[g], 0, 0)),
            pl.BlockSpec((1, HID, DIM), lambda g, be, bc, nt, inv: (be[g], 0, 0)),
            pl.BlockSpec((1, 1, DIM), lambda g, be, bc, nt, inv: (be[g], 0, 0)),
        ],
        out_specs=pl.BlockSpec((T, DIM), lambda g, be, bc, nt, inv: (0, 0)),
        scratch_shapes=[pltpu.VMEM((BLK, DIM), jnp.float32)],
    )
    return pl.pallas_call(
        _ffn_body,
        grid_spec=grid_spec,
        out_shape=jax.ShapeDtypeStruct((T, DIM), jnp.float32),
    )(be, bc, nt, inv, xs, W1, b1.reshape(EXPERTS, 1, HID),
      W2, b2.reshape(EXPERTS, 1, DIM))


def kernel(x, Wr, W1, b1, W2, b2):
    spos, be, bc, nt = _router(x, Wr)

    info = plsc.get_sparse_core_info()
    nc, ns = info.num_cores, info.num_subcores
    nw = nc * ns
    chunk = T // nw
    mesh = plsc.VectorSubcoreMesh(core_axis_name="c", subcore_axis_name="s")

    @functools.partial(
        pl.kernel, mesh=mesh,
        out_type=(
            jax.ShapeDtypeStruct((SLOTS, DIM), jnp.float32),
            jax.ShapeDtypeStruct((SLOTS,), jnp.int32),
        ),
        scratch_types=[
            pltpu.VMEM((chunk,), jnp.int32),
            pltpu.VMEM((chunk,), jnp.int32),
            pltpu.VMEM((chunk, DIM), jnp.float32),
            pltpu.SemaphoreType.DMA,
        ],
    )
    def sc_scatter(x_hbm, spos_hbm, xs_out, inv_out, idx_v, tid_v, rows_v, sem):
        wid = lax.axis_index("s") * nc + lax.axis_index("c")
        base = wid * chunk
        pltpu.sync_copy(spos_hbm.at[pl.ds(base, chunk)], idx_v)
        pltpu.sync_copy(x_hbm.at[pl.ds(base, chunk)], rows_v)
        for k in range(chunk // 16):
            tid_v[pl.ds(k * 16, 16)] = (
                lax.iota(jnp.int32, 16) + (base + k * 16))
        pltpu.async_copy(rows_v, xs_out.at[idx_v], sem).wait()
        pltpu.async_copy(tid_v, inv_out.at[idx_v], sem).wait()

    xs, inv = sc_scatter(x, spos)
    return _ffn(be, bc, nt, inv, xs, W1, b1, W2, b2)
